# baseline (speedup 1.0000x reference)
import functools

import jax
import jax.numpy as jnp
from jax.experimental import pallas as pl
from jax.experimental.pallas import tpu as pltpu

EPS = 1e-5


def _k1(x_ref, adj_ref, w_ref, b_ref, y_ref, s_ref, q_ref, *, bb):
    for i in range(bb):
        s = jnp.dot(x_ref[i], w_ref[...], preferred_element_type=jnp.float32)
        y = jnp.dot(adj_ref[i], s, preferred_element_type=jnp.float32)
        y = jnp.maximum(y + b_ref[...], 0.0)
        y_ref[i] = y.astype(y_ref.dtype)
        s_ref[i] = jnp.sum(y, axis=1, keepdims=True)
        q_ref[i] = jnp.sum(y * y, axis=1, keepdims=True)


def _finalize(s_ref, q_ref, g_ref, be_ref, a_ref, c_ref, count):
    inv = 1.0 / count
    mean = jnp.sum(s_ref[...], axis=0) * inv
    var = jnp.sum(q_ref[...], axis=0) * inv - mean * mean
    a = g_ref[...] * jax.lax.rsqrt(var + EPS)
    a_ref[...] = a
    c_ref[...] = be_ref[...] - mean * a


def _k2(y1_ref, adj_ref, w_ref, b_ref, s1_ref, q1_ref, g_ref, be_ref,
        y2_ref, s_ref, q_ref, a_s, c_s, *, count, bb):
    @pl.when(pl.program_id(0) == 0)
    def _():
        _finalize(s1_ref, q1_ref, g_ref, be_ref, a_s, c_s, count)

    for i in range(bb):
        h = y1_ref[i].astype(jnp.float32) * a_s[...] + c_s[...]
        s2 = jnp.dot(h, w_ref[...], preferred_element_type=jnp.float32)
        y2 = jnp.dot(adj_ref[i], s2, preferred_element_type=jnp.float32)
        y2 = jnp.maximum(y2 + b_ref[...], 0.0)
        y2_ref[i] = y2.astype(y2_ref.dtype)
        s_ref[i] = jnp.sum(y2, axis=1, keepdims=True)
        q_ref[i] = jnp.sum(y2 * y2, axis=1, keepdims=True)


def _k3(y2_ref, s2_ref, q2_ref, g_ref, be_ref, out_ref, a_s, c_s, *, count):
    @pl.when(pl.program_id(0) == 0)
    def _():
        _finalize(s2_ref, q2_ref, g_ref, be_ref, a_s, c_s, count)

    out_ref[...] = (y2_ref[...].astype(jnp.float32) * a_s[...][None]
                    + c_s[...][None])


@jax.jit
def kernel(x, adj, W1, b1, W2, b2, gamma1, beta1, gamma2, beta2):
    B, N, C_in = x.shape
    C_hid = W1.shape[1]
    C_out = W2.shape[1]
    f32 = jnp.float32
    bf16 = jnp.bfloat16

    full = lambda shape: pl.BlockSpec(shape, lambda b: (0,) * len(shape))
    blk = lambda *dims: pl.BlockSpec(dims, lambda b: (b,) + (0,) * (len(dims) - 1))
    stat = jax.ShapeDtypeStruct((B, N, 1), f32)
    vec_scratch = pltpu.VMEM((N, 1), f32)

    BB1 = 4
    y1, s1, q1 = pl.pallas_call(
        functools.partial(_k1, bb=BB1),
        grid=(B // BB1,),
        in_specs=[blk(BB1, N, C_in), blk(BB1, N, N), full((C_in, C_hid)),
                  full((1, C_hid))],
        out_specs=[blk(BB1, N, C_hid), blk(BB1, N, 1), blk(BB1, N, 1)],
        out_shape=[jax.ShapeDtypeStruct((B, N, C_hid), bf16), stat, stat],
    )(x, adj, W1, b1.reshape(1, C_hid))

    BB = 2
    y2, s2, q2 = pl.pallas_call(
        functools.partial(_k2, count=B * C_hid, bb=BB),
        grid=(B // BB,),
        in_specs=[blk(BB, N, C_hid), blk(BB, N, N), full((C_hid, C_out)),
                  full((1, C_out)), full((B, N, 1)), full((B, N, 1)),
                  full((N, 1)), full((N, 1))],
        out_specs=[blk(BB, N, C_out), blk(BB, N, 1), blk(BB, N, 1)],
        out_shape=[jax.ShapeDtypeStruct((B, N, C_out), bf16), stat, stat],
        scratch_shapes=[vec_scratch, vec_scratch],
    )(y1, adj, W2, b2.reshape(1, C_out), s1, q1,
      gamma1.reshape(N, 1), beta1.reshape(N, 1))

    BB3 = 8
    out = pl.pallas_call(
        functools.partial(_k3, count=B * C_out),
        grid=(B // BB3,),
        in_specs=[blk(BB3, N, C_out), full((B, N, 1)), full((B, N, 1)),
                  full((N, 1)), full((N, 1))],
        out_specs=blk(BB3, N, C_out),
        out_shape=jax.ShapeDtypeStruct((B, N, C_out), f32),
        scratch_shapes=[vec_scratch, vec_scratch],
    )(y2, s2, q2, gamma2.reshape(N, 1), beta2.reshape(N, 1))

    return out


# speedup vs baseline: 1.3899x; 1.3899x over previous
import jax
import jax.numpy as jnp
from jax.experimental import pallas as pl
from jax.experimental.pallas import tpu as pltpu

EPS = 1e-5
BB = 2


def _body(x_ref, adj_ref, w1_ref, b1_ref, w2_ref, b2_ref,
          g1_ref, be1_ref, g2_ref, be2_ref, out_ref,
          y1_all, y2_all, s1, q1, s2, q2, a1, c1, a2, c2, *, nb, count):
    i = pl.program_id(0)

    @pl.when(i == 0)
    def _init():
        s1[...] = jnp.zeros_like(s1)
        q1[...] = jnp.zeros_like(q1)
        s2[...] = jnp.zeros_like(s2)
        q2[...] = jnp.zeros_like(q2)

    @pl.when(i < nb)
    def _layer1():
        for ii in range(BB):
            s = jnp.dot(x_ref[ii], w1_ref[...],
                        preferred_element_type=jnp.float32)
            y = jnp.dot(adj_ref[ii], s, preferred_element_type=jnp.float32)
            y = jnp.maximum(y + b1_ref[...], 0.0)
            y1_all[pl.ds(i * BB + ii, 1)] = y[None].astype(y1_all.dtype)
            s1[...] += jnp.sum(y, axis=1, keepdims=True)
            q1[...] += jnp.sum(y * y, axis=1, keepdims=True)

    @pl.when(i == nb)
    def _fin1():
        inv = 1.0 / count
        mean = s1[...] * inv
        var = q1[...] * inv - mean * mean
        a = g1_ref[...] * jax.lax.rsqrt(var + EPS)
        a1[...] = a
        c1[...] = be1_ref[...] - mean * a

    @pl.when((i >= nb) & (i < 2 * nb))
    def _layer2():
        j = i - nb
        for ii in range(BB):
            h = (y1_all[j * BB + ii].astype(jnp.float32) * a1[...]
                 + c1[...])
            s2v = jnp.dot(h, w2_ref[...], preferred_element_type=jnp.float32)
            y = jnp.dot(adj_ref[ii], s2v, preferred_element_type=jnp.float32)
            y = jnp.maximum(y + b2_ref[...], 0.0)
            y2_all[pl.ds(j * BB + ii, 1)] = y[None].astype(y2_all.dtype)
            s2[...] += jnp.sum(y, axis=1, keepdims=True)
            q2[...] += jnp.sum(y * y, axis=1, keepdims=True)

    @pl.when(i == 2 * nb)
    def _fin2():
        inv = 1.0 / count
        mean = s2[...] * inv
        var = q2[...] * inv - mean * mean
        a = g2_ref[...] * jax.lax.rsqrt(var + EPS)
        a2[...] = a
        c2[...] = be2_ref[...] - mean * a

    @pl.when(i >= 2 * nb)
    def _norm():
        j = i - 2 * nb
        for ii in range(BB):
            out_ref[ii] = (y2_all[j * BB + ii].astype(jnp.float32)
                           * a2[...] + c2[...])


@jax.jit
def kernel(x, adj, W1, b1, W2, b2, gamma1, beta1, gamma2, beta2):
    B, N, C_in = x.shape
    C_hid = W1.shape[1]
    C_out = W2.shape[1]
    f32 = jnp.float32
    nb = B // BB
    import functools

    full = lambda shape: pl.BlockSpec(shape, lambda i: (0,) * len(shape))
    vec = pltpu.VMEM((N, 1), f32)

    def adj_idx(i):
        return (jnp.where(i < nb, i, jnp.where(i < 2 * nb, i - nb, nb - 1)),
                0, 0)

    out = pl.pallas_call(
        functools.partial(_body, nb=nb, count=B * C_hid),
        grid=(3 * nb,),
        in_specs=[
            pl.BlockSpec((BB, N, C_in), lambda i: (jnp.minimum(i, nb - 1), 0, 0)),
            pl.BlockSpec((BB, N, N), adj_idx),
            full((C_in, C_hid)), full((1, C_hid)),
            full((C_hid, C_out)), full((1, C_out)),
            full((N, 1)), full((N, 1)), full((N, 1)), full((N, 1)),
        ],
        out_specs=pl.BlockSpec(
            (BB, N, C_out), lambda i: (jnp.maximum(i - 2 * nb, 0), 0, 0)),
        out_shape=jax.ShapeDtypeStruct((B, N, C_out), f32),
        scratch_shapes=[
            pltpu.VMEM((B, N, C_hid), jnp.bfloat16),
            pltpu.VMEM((B, N, C_out), jnp.bfloat16),
            vec, vec, vec, vec, vec, vec, vec, vec,
        ],
    )(x, adj, W1, b1.reshape(1, C_hid), W2, b2.reshape(1, C_out),
      gamma1.reshape(N, 1), beta1.reshape(N, 1),
      gamma2.reshape(N, 1), beta2.reshape(N, 1))

    return out
